# Initial kernel scaffold; baseline (speedup 1.0000x reference)
import jax
import jax.numpy as jnp
from jax.experimental import pallas as pl


def kernel(ROIs, probs, deltas):
    raise NotImplementedError("write your pallas kernel here")


# speedup vs baseline: 216.6112x; 216.6112x over previous
import functools

import jax
import jax.numpy as jnp
import numpy as np
from jax import lax
from jax.experimental import pallas as pl
from jax.experimental.pallas import tpu as pltpu
from jax.experimental.pallas import tpu_sc as plsc

N = 20000
C = 81
MAXI = 100
MINCONF = 0.7
NMS_T = 0.3
BLK = 2000
NSUB = 16
NCLS = 5
KCAP = 112
SENT = np.int32(0x7FFFFFFF)
BIGP = np.int32(1 << 30)


def _prep_body(rois_ref, probs_ref, deltas_ref, boxes_ref, dkey_ref, ccid_ref):
    p = probs_ref[...]
    m = jnp.max(p, axis=1, keepdims=True)
    iot = lax.broadcasted_iota(jnp.int32, p.shape, 1)
    cid = jnp.min(jnp.where(p == m, iot, C), axis=1)
    score = m[:, 0]

    d = deltas_ref[...]
    j = lax.broadcasted_iota(jnp.int32, d.shape, 1)
    selc = (j >> 2) == cid[:, None]
    comp = j & 3
    dy = jnp.sum(jnp.where(selc & (comp == 0), d, 0.0), axis=1) * 0.1
    dx = jnp.sum(jnp.where(selc & (comp == 1), d, 0.0), axis=1) * 0.1
    dh = jnp.sum(jnp.where(selc & (comp == 2), d, 0.0), axis=1) * 0.2
    dw = jnp.sum(jnp.where(selc & (comp == 3), d, 0.0), axis=1) * 0.2

    r = rois_ref[...]
    h = r[:, 2] - r[:, 0]
    w = r[:, 3] - r[:, 1]
    cy = (r[:, 0] + 0.5 * h) + dy * h
    cx = (r[:, 1] + 0.5 * w) + dx * w
    h = h * jnp.exp(dh)
    w = w * jnp.exp(dw)
    y1 = cy - 0.5 * h
    x1 = cx - 0.5 * w
    y2 = y1 + h
    x2 = x1 + w
    y1c = jnp.clip(y1, 0.0, 1.0)
    x1c = jnp.clip(x1, 0.0, 1.0)
    y2c = jnp.clip(y2, 0.0, 1.0)
    x2c = jnp.clip(x2, 0.0, 1.0)
    boxes_ref[...] = jnp.concatenate(
        [y1c[:, None], x1c[:, None], y2c[:, None], x2c[:, None],
         jnp.zeros((y1c.shape[0], 124), jnp.float32)], axis=1)

    dkey_ref[...] = jnp.bitwise_xor(
        lax.bitcast_convert_type(score, jnp.int32), jnp.int32(-1))[:, None]
    cand = (cid > 0) & (score >= MINCONF)
    ccid_ref[...] = jnp.where(cand, cid, 0)[:, None]


def _prep(rois, probs, deltas):
    nb = N // BLK
    return pl.pallas_call(
        _prep_body,
        grid=(nb,),
        in_specs=[
            pl.BlockSpec((BLK, 4), lambda i: (i, 0)),
            pl.BlockSpec((BLK, C), lambda i: (i, 0)),
            pl.BlockSpec((BLK, 4 * C), lambda i: (i, 0)),
        ],
        out_specs=[
            pl.BlockSpec((BLK, 128), lambda i: (i, 0)),
            pl.BlockSpec((BLK, 1), lambda i: (i, 0)),
            pl.BlockSpec((BLK, 1), lambda i: (i, 0)),
        ],
        out_shape=[
            jax.ShapeDtypeStruct((N, 128), jnp.float32),
            jax.ShapeDtypeStruct((N, 1), jnp.int32),
            jax.ShapeDtypeStruct((N, 1), jnp.int32),
        ],
    )(rois, probs, deltas)


def _iota16():
    return lax.broadcasted_iota(jnp.int32, (16,), 0)


def _sc_body(dkey_hbm, ccid_hbm, boxes_hbm, out_hbm,
             keyv, cidv, ak, ai, hist, kg, ka, b16, i16, mrowf, scnt, sem):
    w = lax.axis_index("s")
    z16i = jnp.zeros((16,), jnp.int32)
    z16f = jnp.zeros((16,), jnp.float32)
    lane0 = _iota16() == 0

    def _sget(ref, off):
        return ref[pl.ds(off, 16)][0]

    def _sput(ref, off, val, dtype=jnp.int32):
        plsc.store_compressed(ref.at[pl.ds(off, 16)],
                              jnp.full((16,), val, dtype), mask=lane0)

    def _sputsel(ref, off, cond, a, b):
        m16 = jnp.full((16,), cond)
        v = jnp.where(m16, jnp.full((16,), a, jnp.int32),
                      jnp.full((16,), b, jnp.int32))
        plsc.store_compressed(ref.at[pl.ds(off, 16)], v, mask=lane0)

    pltpu.sync_copy(dkey_hbm, keyv.at[pl.ds(0, N)])
    pltpu.sync_copy(ccid_hbm, cidv.at[pl.ds(0, N)])

    def _init_scnt(s, _):
        scnt[s] = 0
        return 0
    lax.fori_loop(0, 16, _init_scnt, 0)

    nine = jnp.full((16,), 9.0, jnp.float32)

    def _init_kg(i, _):
        kg[pl.ds(i * 16, 16)] = nine
        return 0
    lax.fori_loop(0, 4 * NCLS * (KCAP // 16), _init_kg, 0)

    def _init_ka(i, _):
        ka[pl.ds(i * 16, 16)] = z16f
        return 0
    lax.fori_loop(0, NCLS * (KCAP // 16), _init_ka, 0)

    def _scan(ch, cur):
        cv = cidv[pl.ds(ch * 16, 16)]
        kv = keyv[pl.ds(ch * 16, 16)]
        mask = (cv != 0) & (((cv - 1) & (NSUB - 1)) == w)
        gidx = ch * 16 + _iota16()
        packed = (gidx << 7) | cv
        plsc.store_compressed(ak.at[pl.ds(cur, 16)], kv, mask=mask)
        plsc.store_compressed(ai.at[pl.ds(cur, 16)], packed, mask=mask)
        return cur + jnp.sum(mask.astype(jnp.int32))

    K = lax.fori_loop(0, N // 16, _scan, jnp.int32(0))

    def _pass(shift, src_k, src_i, dst_k, dst_i):
        def _zh(b, _):
            hist[pl.ds(b * 16, 16)] = z16i
            return 0
        lax.fori_loop(0, 17, _zh, 0)

        def _h(t, _):
            dd = lax.shift_right_logical(_sget(src_k, t), shift) & 255
            _sput(hist, dd, _sget(hist, dd) + 1)
            return 0
        lax.fori_loop(0, K, _h, 0)

        def _pf(b, run):
            t = _sget(hist, b)
            _sput(hist, b, run)
            return run + t
        lax.fori_loop(0, 256, _pf, jnp.int32(0))

        def _sct(t, _):
            kk = _sget(src_k, t)
            dd = lax.shift_right_logical(kk, shift) & 255
            p = _sget(hist, dd)
            _sput(hist, dd, p + 1)
            _sput(dst_k, p, kk)
            _sput(dst_i, p, _sget(src_i, t))
            return 0
        lax.fori_loop(0, K, _sct, 0)

    _pass(0, ak, ai, keyv, cidv)
    _pass(8, keyv, cidv, ak, ai)
    _pass(16, ak, ai, keyv, cidv)
    _pass(24, keyv, cidv, ak, ai)

    ai[pl.ds(K, 16)] = z16i

    col4 = _iota16() & 3

    @pl.loop(0, (K + 15) >> 4)
    def _chunk(ch):
        i16[...] = lax.shift_right_logical(ai[pl.ds(ch * 16, 16)], 7)
        pltpu.async_copy(boxes_hbm.at[i16], b16, sem).wait()
        lim = jnp.minimum(16, K - ch * 16)

        def _elem(l, _):
            t = ch * 16 + l
            p = _sget(ai, t)
            c = p & 127
            s = (c - 1) >> 4
            cs = scnt[s]

            @pl.when(cs < MAXI)
            def _():
                g4 = plsc.load_gather(b16, [jnp.full((16,), l, jnp.int32), col4])
                y1 = g4[0]
                x1 = g4[1]
                y2 = g4[2]
                x2 = g4[3]
                ar = (y2 - y1) * (x2 - x1)
                nj = (cs + 15) >> 4

                def _iou(j, acc):
                    o = s * KCAP + j * 16
                    ky1 = kg[pl.ds(o, 16)]
                    kx1 = kg[pl.ds(560 + o, 16)]
                    ky2 = kg[pl.ds(1120 + o, 16)]
                    kx2 = kg[pl.ds(1680 + o, 16)]
                    kar = ka[pl.ds(o, 16)]
                    yy1 = jnp.maximum(y1, ky1)
                    xx1 = jnp.maximum(x1, kx1)
                    yy2 = jnp.minimum(y2, ky2)
                    xx2 = jnp.minimum(x2, kx2)
                    inter = jnp.maximum(0.0, yy2 - yy1) * jnp.maximum(0.0, xx2 - xx1)
                    union = ar + kar - inter
                    iou = inter / jnp.maximum(union, 1e-12)
                    return acc | jnp.any(iou > NMS_T)

                sup = lax.fori_loop(0, nj, _iou, False)

                @pl.when(jnp.logical_not(sup))
                def _():
                    o = s * KCAP + cs
                    _sput(kg, o, y1, jnp.float32)
                    _sput(kg, 560 + o, x1, jnp.float32)
                    _sput(kg, 1120 + o, y2, jnp.float32)
                    _sput(kg, 1680 + o, x2, jnp.float32)
                    _sput(ka, o, ar, jnp.float32)
                    scnt[s] = cs + 1
                    mc = scnt[6]
                    _sput(mrowf, KOFF + mc, _sget(ak, t))
                    _sput(mrowf, GOFF + mc, p)
                    scnt[6] = mc + 1
            return 0

        lax.fori_loop(0, lim, _elem, 0)

    _sput(mrowf, 0, scnt[6])
    pltpu.sync_copy(mrowf.at[pl.ds(0, ROWL)],
                    out_hbm.at[pl.ds(w * ROWL, ROWL)])


def _sc_merge_body(lists_hbm, boxes_hbm, out_hbm,
                   allv, hk, hg, selg, selk, selc, boxr, outv, scnt, ptr, sem):
    ROW = ROWL
    w = lax.axis_index("s")
    z16f = jnp.zeros((16,), jnp.float32)
    lane0 = _iota16() == 0

    def _sput(ref, off, val, dtype=jnp.int32):
        plsc.store_compressed(ref.at[pl.ds(off, 16)],
                              jnp.full((16,), val, dtype), mask=lane0)

    def _ip(s, _):
        ptr[s] = 0
        scnt[s] = 0
        return 0
    lax.fori_loop(0, 16, _ip, 0)

    def _zo(b, _):
        outv[pl.ds(b * 16, 16)] = z16f
        return 0
    lax.fori_loop(0, 39, _zo, 0)

    @pl.when(w == 0)
    def _():
        pltpu.sync_copy(lists_hbm, allv.at[pl.ds(0, 16 * ROW)])

        def _ih(s, _):
            cl = allv[pl.ds(s * ROW, 16)][0]
            m16 = jnp.full((16,), cl > 0)
            kv = jnp.where(m16,
                           jnp.full((16,), allv[pl.ds(s * ROW + KOFF, 16)][0]),
                           jnp.full((16,), SENT))
            gv = jnp.where(m16,
                           jnp.full((16,), allv[pl.ds(s * ROW + GOFF, 16)][0]),
                           jnp.full((16,), BIGP))
            plsc.store_compressed(hk.at[pl.ds(s, 16)], kv, mask=lane0)
            plsc.store_compressed(hg.at[pl.ds(s, 16)], gv, mask=lane0)
            return 0
        lax.fori_loop(0, 16, _ih, 0)

        def _zs(b, _):
            selg[pl.ds(b * 16, 16)] = b * 16 + _iota16()
            return 0
        lax.fori_loop(0, 8, _zs, 0)

        def _pick(t, _):
            hkv = hk[pl.ds(0, 16)]
            m = jnp.min(hkv)

            @pl.when(m != SENT)
            def _():
                hgv = hg[pl.ds(0, 16)]
                cnd = jnp.where(hkv == m, hgv, jnp.full((16,), BIGP))
                gp = jnp.min(cnd)
                l = plsc.all_reduce_ffs(cnd == gp)[0]
                g = lax.shift_right_logical(gp, 7)
                c = gp & 127
                _sput(selg, t, g)
                _sput(selk, t, jnp.bitwise_xor(m, jnp.int32(-1)))
                _sput(selc, t, c)
                scnt[7] = t + 1
                pp = ptr[l] + 1
                ptr[l] = pp
                cl = allv[pl.ds(l * ROW, 16)][0]
                nk = allv[pl.ds(l * ROW + KOFF + pp, 16)][0]
                ng = allv[pl.ds(l * ROW + GOFF + pp, 16)][0]
                m16 = jnp.full((16,), pp < cl)
                kv = jnp.where(m16, jnp.full((16,), nk), jnp.full((16,), SENT))
                gv = jnp.where(m16, jnp.full((16,), ng), jnp.full((16,), BIGP))
                plsc.store_compressed(hk.at[pl.ds(l, 16)], kv, mask=lane0)
                plsc.store_compressed(hg.at[pl.ds(l, 16)], gv, mask=lane0)
            return 0

        lax.fori_loop(0, MAXI, _pick, 0)

        def _conv(t, _):
            sc16 = selc[pl.ds(t, 16)].astype(jnp.float32)
            plsc.store_compressed(outv.at[pl.ds(t * 6 + 4, 16)], sc16,
                                  mask=lane0)
            sk16 = plsc.bitcast(selk[pl.ds(t, 16)], jnp.float32)
            plsc.store_compressed(outv.at[pl.ds(t * 6 + 5, 16)], sk16,
                                  mask=lane0)
            return 0
        lax.fori_loop(0, scnt[7], _conv, 0)

        pltpu.async_copy(boxes_hbm.at[selg], boxr, sem).wait()
        col4 = _iota16() & 3

        def _fill(t, _):
            b4 = plsc.load_gather(boxr, [jnp.full((16,), t, jnp.int32), col4])
            for k in range(4):
                _sput(outv, t * 6 + k, b4[k], jnp.float32)
            return 0
        lax.fori_loop(0, scnt[7], _fill, 0)

        pltpu.sync_copy(outv.at[pl.ds(0, 600)], out_hbm)


KOFF = 8
GOFF = 584
ROWL = 1152


@functools.cache
def _make_sc_nms():
    cp = pltpu.CompilerParams(needs_layout_passes=False,
                              use_tc_tiling_on_sc=False)
    mesh = plsc.VectorSubcoreMesh(core_axis_name="c", subcore_axis_name="s",
                                  num_cores=1)
    nms = pl.kernel(
        _sc_body,
        out_type=jax.ShapeDtypeStruct((16 * ROWL,), jnp.int32),
        mesh=mesh,
        compiler_params=cp,
        scratch_types=[
            pltpu.VMEM((N + 16,), jnp.int32),
            pltpu.VMEM((N + 16,), jnp.int32),
            pltpu.VMEM((N + 16,), jnp.int32),
            pltpu.VMEM((N + 16,), jnp.int32),
            pltpu.VMEM((272,), jnp.int32),
            pltpu.VMEM((2256,), jnp.float32),
            pltpu.VMEM((576,), jnp.float32),
            pltpu.VMEM((16, 128), jnp.float32),
            pltpu.VMEM((16,), jnp.int32),
            pltpu.VMEM((ROWL + 16,), jnp.int32),
            pltpu.SMEM((16,), jnp.int32),
            pltpu.SemaphoreType.DMA,
        ])
    merge = pl.kernel(
        _sc_merge_body,
        out_type=jax.ShapeDtypeStruct((600,), jnp.float32),
        mesh=mesh,
        compiler_params=cp,
        scratch_types=[
            pltpu.VMEM((16 * ROWL + 16,), jnp.int32),
            pltpu.VMEM((32,), jnp.int32),
            pltpu.VMEM((32,), jnp.int32),
            pltpu.VMEM((128,), jnp.int32),
            pltpu.VMEM((128,), jnp.int32),
            pltpu.VMEM((128,), jnp.int32),
            pltpu.VMEM((128, 128), jnp.float32),
            pltpu.VMEM((624,), jnp.float32),
            pltpu.SMEM((16,), jnp.int32),
            pltpu.SMEM((16,), jnp.int32),
            pltpu.SemaphoreType.DMA,
        ])
    return nms, merge


def kernel(ROIs, probs, deltas):
    rois = ROIs.reshape(N, 4)
    p = probs.reshape(N, C)
    d = deltas.reshape(N, 4 * C)
    boxes, dkey, ccid = _prep(rois, p, d)
    nms, merge = _make_sc_nms()
    lists = nms(dkey.reshape(N), ccid.reshape(N), boxes)
    out = merge(lists, boxes)
    return out.reshape(1, MAXI, 6)


# speedup vs baseline: 290.6006x; 1.3416x over previous
import functools

import jax
import jax.numpy as jnp
import numpy as np
from jax import lax
from jax.experimental import pallas as pl
from jax.experimental.pallas import tpu as pltpu
from jax.experimental.pallas import tpu_sc as plsc

N = 20000
C = 81
MAXI = 100
MINCONF = 0.7
NMS_T = 0.3
BLK = 2000
NSUB = 16
NCLS = 5
KCAP = 112
SENT = np.int32(0x7FFFFFFF)
BIGP = np.int32(1 << 30)


def _prep_body(rois_ref, probs_ref, deltas_ref, boxes_ref, dkey_ref, ccid_ref):
    p = probs_ref[...]
    m = jnp.max(p, axis=1, keepdims=True)
    iot = lax.broadcasted_iota(jnp.int32, p.shape, 1)
    cid = jnp.min(jnp.where(p == m, iot, C), axis=1)
    score = m[:, 0]

    d = deltas_ref[...]
    j = lax.broadcasted_iota(jnp.int32, d.shape, 1)
    selc = (j >> 2) == cid[:, None]
    comp = j & 3
    dy = jnp.sum(jnp.where(selc & (comp == 0), d, 0.0), axis=1) * 0.1
    dx = jnp.sum(jnp.where(selc & (comp == 1), d, 0.0), axis=1) * 0.1
    dh = jnp.sum(jnp.where(selc & (comp == 2), d, 0.0), axis=1) * 0.2
    dw = jnp.sum(jnp.where(selc & (comp == 3), d, 0.0), axis=1) * 0.2

    r = rois_ref[...]
    h = r[:, 2] - r[:, 0]
    w = r[:, 3] - r[:, 1]
    cy = (r[:, 0] + 0.5 * h) + dy * h
    cx = (r[:, 1] + 0.5 * w) + dx * w
    h = h * jnp.exp(dh)
    w = w * jnp.exp(dw)
    y1 = cy - 0.5 * h
    x1 = cx - 0.5 * w
    y2 = y1 + h
    x2 = x1 + w
    y1c = jnp.clip(y1, 0.0, 1.0)
    x1c = jnp.clip(x1, 0.0, 1.0)
    y2c = jnp.clip(y2, 0.0, 1.0)
    x2c = jnp.clip(x2, 0.0, 1.0)
    boxes_ref[...] = jnp.concatenate(
        [y1c[:, None], x1c[:, None], y2c[:, None], x2c[:, None],
         jnp.zeros((y1c.shape[0], 124), jnp.float32)], axis=1)

    dkey_ref[...] = jnp.bitwise_xor(
        lax.bitcast_convert_type(score, jnp.int32), jnp.int32(-1))[:, None]
    cand = (cid > 0) & (score >= MINCONF)
    ccid_ref[...] = jnp.where(cand, cid, 0)[:, None]


def _prep(rois, probs, deltas):
    nb = N // BLK
    return pl.pallas_call(
        _prep_body,
        grid=(nb,),
        in_specs=[
            pl.BlockSpec((BLK, 4), lambda i: (i, 0)),
            pl.BlockSpec((BLK, C), lambda i: (i, 0)),
            pl.BlockSpec((BLK, 4 * C), lambda i: (i, 0)),
        ],
        out_specs=[
            pl.BlockSpec((BLK, 128), lambda i: (i, 0)),
            pl.BlockSpec((BLK, 1), lambda i: (i, 0)),
            pl.BlockSpec((BLK, 1), lambda i: (i, 0)),
        ],
        out_shape=[
            jax.ShapeDtypeStruct((N, 128), jnp.float32),
            jax.ShapeDtypeStruct((N, 1), jnp.int32),
            jax.ShapeDtypeStruct((N, 1), jnp.int32),
        ],
    )(rois, probs, deltas)


def _iota16():
    return lax.broadcasted_iota(jnp.int32, (16,), 0)


def _sc_body(dkey_hbm, ccid_hbm, boxes_hbm, out_hbm,
             keyv, cidv, ak, ai, hist, hist2, kg, ka, b16, i16, mrowf,
             scnt, sem):
    w = lax.axis_index("s")
    z16i = jnp.zeros((16,), jnp.int32)
    z16f = jnp.zeros((16,), jnp.float32)
    lane0 = _iota16() == 0

    def _sget(ref, off):
        return ref[pl.ds(off, 16)][0]

    def _sput(ref, off, val, dtype=jnp.int32):
        plsc.store_compressed(ref.at[pl.ds(off, 16)],
                              jnp.full((16,), val, dtype), mask=lane0)

    def _sputsel(ref, off, cond, a, b):
        m16 = jnp.full((16,), cond)
        v = jnp.where(m16, jnp.full((16,), a, jnp.int32),
                      jnp.full((16,), b, jnp.int32))
        plsc.store_compressed(ref.at[pl.ds(off, 16)], v, mask=lane0)

    pltpu.sync_copy(dkey_hbm, keyv.at[pl.ds(0, N)])
    pltpu.sync_copy(ccid_hbm, cidv.at[pl.ds(0, N)])

    def _init_scnt(s, _):
        scnt[s] = 0
        return 0
    lax.fori_loop(0, 16, _init_scnt, 0)

    nine = jnp.full((16,), 9.0, jnp.float32)

    def _init_kg(i, _):
        kg[pl.ds(i * 16, 16)] = nine
        return 0
    lax.fori_loop(0, 4 * NCLS * (KCAP // 16), _init_kg, 0)

    def _init_ka(i, _):
        ka[pl.ds(i * 16, 16)] = z16f
        return 0
    lax.fori_loop(0, NCLS * (KCAP // 16), _init_ka, 0)

    def _scan(ch, cur):
        cv = cidv[pl.ds(ch * 16, 16)]
        kv = keyv[pl.ds(ch * 16, 16)]
        mask = (cv != 0) & (((cv - 1) & (NSUB - 1)) == w)
        gidx = ch * 16 + _iota16()
        packed = (gidx << 7) | cv
        plsc.store_compressed(ak.at[pl.ds(cur, 16)], kv, mask=mask)
        plsc.store_compressed(ai.at[pl.ds(cur, 16)], packed, mask=mask)
        return cur + jnp.sum(mask.astype(jnp.int32))

    K = lax.fori_loop(0, N // 16, _scan, jnp.int32(0))

    def _pass(shift, src_k, src_i, dst_k, dst_i):
        def _zh2(i, _):
            l = i >> 4
            b = i & 15
            hist2[l, pl.ds(b * 16, 16)] = z16i
            return 0
        lax.fori_loop(0, 256, _zh2, 0)

        ones16 = jnp.full((16,), 1, jnp.int32)

        def _hv(g, _):
            kv = src_k[pl.ds(g * 16, 16)]
            dv = lax.shift_right_logical(kv, shift) & 255
            plsc.addupdate_scatter(hist2, [_iota16(), dv], ones16)
            return 0
        lax.fori_loop(0, K >> 4, _hv, 0)

        def _ht(t, _):
            dd = lax.shift_right_logical(_sget(src_k, t), shift) & 255
            plsc.addupdate_scatter(hist2, [z16i, jnp.full((16,), dd)],
                                   ones16, mask=lane0)
            return 0
        lax.fori_loop(K & ~15, K, _ht, 0)

        def _tot(c, _):
            def _acc(l, acc):
                return acc + hist2[l, pl.ds(c * 16, 16)]
            hist[pl.ds(c * 16, 16)] = lax.fori_loop(0, 16, _acc, z16i)
            return 0
        lax.fori_loop(0, 16, _tot, 0)

        def _pf(b, run):
            t = _sget(hist, b)
            _sput(hist, b, run)
            return run + t
        lax.fori_loop(0, 256, _pf, jnp.int32(0))

        def _sct(t, _):
            kk = _sget(src_k, t)
            dd = lax.shift_right_logical(kk, shift) & 255
            p = _sget(hist, dd)
            _sput(hist, dd, p + 1)
            _sput(dst_k, p, kk)
            _sput(dst_i, p, _sget(src_i, t))
            return 0
        lax.fori_loop(0, K, _sct, 0)

    _pass(0, ak, ai, keyv, cidv)
    _pass(8, keyv, cidv, ak, ai)
    _pass(16, ak, ai, keyv, cidv)
    sk, si = keyv, cidv

    si[pl.ds(K, 16)] = z16i

    col4 = _iota16() & 3

    @pl.loop(0, (K + 15) >> 4)
    def _chunk(ch):
        i16[...] = lax.shift_right_logical(si[pl.ds(ch * 16, 16)], 7)
        pltpu.async_copy(boxes_hbm.at[i16], b16, sem).wait()
        lim = jnp.minimum(16, K - ch * 16)

        def _elem(l, _):
            t = ch * 16 + l
            p = _sget(si, t)
            c = p & 127
            s = (c - 1) >> 4
            cs = scnt[s]

            @pl.when(cs < MAXI)
            def _():
                g4 = plsc.load_gather(b16, [jnp.full((16,), l, jnp.int32), col4])
                y1 = g4[0]
                x1 = g4[1]
                y2 = g4[2]
                x2 = g4[3]
                ar = (y2 - y1) * (x2 - x1)
                nj = (cs + 15) >> 4

                def _iou(j, acc):
                    o = s * KCAP + j * 16
                    ky1 = kg[pl.ds(o, 16)]
                    kx1 = kg[pl.ds(560 + o, 16)]
                    ky2 = kg[pl.ds(1120 + o, 16)]
                    kx2 = kg[pl.ds(1680 + o, 16)]
                    kar = ka[pl.ds(o, 16)]
                    yy1 = jnp.maximum(y1, ky1)
                    xx1 = jnp.maximum(x1, kx1)
                    yy2 = jnp.minimum(y2, ky2)
                    xx2 = jnp.minimum(x2, kx2)
                    inter = jnp.maximum(0.0, yy2 - yy1) * jnp.maximum(0.0, xx2 - xx1)
                    union = ar + kar - inter
                    iou = inter / jnp.maximum(union, 1e-12)
                    return acc | jnp.any(iou > NMS_T)

                sup = lax.fori_loop(0, nj, _iou, False)

                @pl.when(jnp.logical_not(sup))
                def _():
                    o = s * KCAP + cs
                    _sput(kg, o, y1, jnp.float32)
                    _sput(kg, 560 + o, x1, jnp.float32)
                    _sput(kg, 1120 + o, y2, jnp.float32)
                    _sput(kg, 1680 + o, x2, jnp.float32)
                    _sput(ka, o, ar, jnp.float32)
                    scnt[s] = cs + 1
                    mc = scnt[6]
                    _sput(mrowf, KOFF + mc, _sget(sk, t))
                    _sput(mrowf, GOFF + mc, p)
                    scnt[6] = mc + 1
            return 0

        lax.fori_loop(0, lim, _elem, 0)

    _sput(mrowf, 0, scnt[6])
    pltpu.sync_copy(mrowf.at[pl.ds(0, ROWL)],
                    out_hbm.at[pl.ds(w * ROWL, ROWL)])


def _sc_merge_body(lists_hbm, boxes_hbm, out_hbm,
                   allv, hk, hg, selg, selk, selc, boxr, outv, scnt, ptr, sem):
    ROW = ROWL
    w = lax.axis_index("s")
    z16f = jnp.zeros((16,), jnp.float32)
    lane0 = _iota16() == 0

    def _sput(ref, off, val, dtype=jnp.int32):
        plsc.store_compressed(ref.at[pl.ds(off, 16)],
                              jnp.full((16,), val, dtype), mask=lane0)

    def _ip(s, _):
        ptr[s] = 0
        scnt[s] = 0
        return 0
    lax.fori_loop(0, 16, _ip, 0)

    def _zo(b, _):
        outv[pl.ds(b * 16, 16)] = z16f
        return 0
    lax.fori_loop(0, 39, _zo, 0)

    @pl.when(w == 0)
    def _():
        pltpu.sync_copy(lists_hbm, allv.at[pl.ds(0, 16 * ROW)])

        def _ih(s, _):
            cl = allv[pl.ds(s * ROW, 16)][0]
            m16 = jnp.full((16,), cl > 0)
            kv = jnp.where(m16,
                           jnp.full((16,), allv[pl.ds(s * ROW + KOFF, 16)][0]),
                           jnp.full((16,), SENT))
            gv = jnp.where(m16,
                           jnp.full((16,), allv[pl.ds(s * ROW + GOFF, 16)][0]),
                           jnp.full((16,), BIGP))
            plsc.store_compressed(hk.at[pl.ds(s, 16)], kv, mask=lane0)
            plsc.store_compressed(hg.at[pl.ds(s, 16)], gv, mask=lane0)
            return 0
        lax.fori_loop(0, 16, _ih, 0)

        def _zs(b, _):
            selg[pl.ds(b * 16, 16)] = b * 16 + _iota16()
            return 0
        lax.fori_loop(0, 8, _zs, 0)

        def _pick(t, _):
            hkv = hk[pl.ds(0, 16)]
            m = jnp.min(hkv)

            @pl.when(m != SENT)
            def _():
                hgv = hg[pl.ds(0, 16)]
                cnd = jnp.where(hkv == m, hgv, jnp.full((16,), BIGP))
                gp = jnp.min(cnd)
                l = plsc.all_reduce_ffs(cnd == gp)[0]
                g = lax.shift_right_logical(gp, 7)
                c = gp & 127
                _sput(selg, t, g)
                _sput(selk, t, jnp.bitwise_xor(m, jnp.int32(-1)))
                _sput(selc, t, c)
                scnt[7] = t + 1
                pp = ptr[l] + 1
                ptr[l] = pp
                cl = allv[pl.ds(l * ROW, 16)][0]
                nk = allv[pl.ds(l * ROW + KOFF + pp, 16)][0]
                ng = allv[pl.ds(l * ROW + GOFF + pp, 16)][0]
                m16 = jnp.full((16,), pp < cl)
                kv = jnp.where(m16, jnp.full((16,), nk), jnp.full((16,), SENT))
                gv = jnp.where(m16, jnp.full((16,), ng), jnp.full((16,), BIGP))
                plsc.store_compressed(hk.at[pl.ds(l, 16)], kv, mask=lane0)
                plsc.store_compressed(hg.at[pl.ds(l, 16)], gv, mask=lane0)
            return 0

        lax.fori_loop(0, MAXI, _pick, 0)

        def _conv(t, _):
            sc16 = selc[pl.ds(t, 16)].astype(jnp.float32)
            plsc.store_compressed(outv.at[pl.ds(t * 6 + 4, 16)], sc16,
                                  mask=lane0)
            sk16 = plsc.bitcast(selk[pl.ds(t, 16)], jnp.float32)
            plsc.store_compressed(outv.at[pl.ds(t * 6 + 5, 16)], sk16,
                                  mask=lane0)
            return 0
        lax.fori_loop(0, scnt[7], _conv, 0)

        pltpu.async_copy(boxes_hbm.at[selg], boxr, sem).wait()
        col4 = _iota16() & 3

        def _fill(t, _):
            b4 = plsc.load_gather(boxr, [jnp.full((16,), t, jnp.int32), col4])
            for k in range(4):
                _sput(outv, t * 6 + k, b4[k], jnp.float32)
            return 0
        lax.fori_loop(0, scnt[7], _fill, 0)

        pltpu.sync_copy(outv.at[pl.ds(0, 600)], out_hbm)


KOFF = 8
GOFF = 584
ROWL = 1152


@functools.cache
def _make_sc_nms():
    cp = pltpu.CompilerParams(needs_layout_passes=False,
                              use_tc_tiling_on_sc=False)
    mesh = plsc.VectorSubcoreMesh(core_axis_name="c", subcore_axis_name="s",
                                  num_cores=1)
    nms = pl.kernel(
        _sc_body,
        out_type=jax.ShapeDtypeStruct((16 * ROWL,), jnp.int32),
        mesh=mesh,
        compiler_params=cp,
        scratch_types=[
            pltpu.VMEM((N + 16,), jnp.int32),
            pltpu.VMEM((N + 16,), jnp.int32),
            pltpu.VMEM((N + 16,), jnp.int32),
            pltpu.VMEM((N + 16,), jnp.int32),
            pltpu.VMEM((272,), jnp.int32),
            pltpu.VMEM((16, 272), jnp.int32),
            pltpu.VMEM((2256,), jnp.float32),
            pltpu.VMEM((576,), jnp.float32),
            pltpu.VMEM((16, 128), jnp.float32),
            pltpu.VMEM((16,), jnp.int32),
            pltpu.VMEM((ROWL + 16,), jnp.int32),
            pltpu.SMEM((16,), jnp.int32),
            pltpu.SemaphoreType.DMA,
        ])
    merge = pl.kernel(
        _sc_merge_body,
        out_type=jax.ShapeDtypeStruct((600,), jnp.float32),
        mesh=mesh,
        compiler_params=cp,
        scratch_types=[
            pltpu.VMEM((16 * ROWL + 16,), jnp.int32),
            pltpu.VMEM((32,), jnp.int32),
            pltpu.VMEM((32,), jnp.int32),
            pltpu.VMEM((128,), jnp.int32),
            pltpu.VMEM((128,), jnp.int32),
            pltpu.VMEM((128,), jnp.int32),
            pltpu.VMEM((128, 128), jnp.float32),
            pltpu.VMEM((624,), jnp.float32),
            pltpu.SMEM((16,), jnp.int32),
            pltpu.SMEM((16,), jnp.int32),
            pltpu.SemaphoreType.DMA,
        ])
    return nms, merge


def kernel(ROIs, probs, deltas):
    rois = ROIs.reshape(N, 4)
    p = probs.reshape(N, C)
    d = deltas.reshape(N, 4 * C)
    boxes, dkey, ccid = _prep(rois, p, d)
    nms, merge = _make_sc_nms()
    lists = nms(dkey.reshape(N), ccid.reshape(N), boxes)
    out = merge(lists, boxes)
    return out.reshape(1, MAXI, 6)


# speedup vs baseline: 373.8417x; 1.2864x over previous
import functools

import jax
import jax.numpy as jnp
import numpy as np
from jax import lax
from jax.experimental import pallas as pl
from jax.experimental.pallas import tpu as pltpu
from jax.experimental.pallas import tpu_sc as plsc

N = 20000
C = 81
MAXI = 100
MINCONF = 0.7
NMS_T = 0.3
BLK = 2000
NSUB = 16
NCLS = 5
KCAP = 112
SENT = np.int32(0x7FFFFFFF)
BIGP = np.int32(1 << 30)


def _prep_body(rois_ref, probs_ref, deltas_ref, boxes_ref, dkey_ref, ccid_ref):
    p = probs_ref[...]
    m = jnp.max(p, axis=1, keepdims=True)
    iot = lax.broadcasted_iota(jnp.int32, p.shape, 1)
    cid = jnp.min(jnp.where(p == m, iot, C), axis=1)
    score = m[:, 0]

    d = deltas_ref[...]
    j = lax.broadcasted_iota(jnp.int32, d.shape, 1)
    selc = (j >> 2) == cid[:, None]
    comp = j & 3
    dy = jnp.sum(jnp.where(selc & (comp == 0), d, 0.0), axis=1) * 0.1
    dx = jnp.sum(jnp.where(selc & (comp == 1), d, 0.0), axis=1) * 0.1
    dh = jnp.sum(jnp.where(selc & (comp == 2), d, 0.0), axis=1) * 0.2
    dw = jnp.sum(jnp.where(selc & (comp == 3), d, 0.0), axis=1) * 0.2

    r = rois_ref[...]
    h = r[:, 2] - r[:, 0]
    w = r[:, 3] - r[:, 1]
    cy = (r[:, 0] + 0.5 * h) + dy * h
    cx = (r[:, 1] + 0.5 * w) + dx * w
    h = h * jnp.exp(dh)
    w = w * jnp.exp(dw)
    y1 = cy - 0.5 * h
    x1 = cx - 0.5 * w
    y2 = y1 + h
    x2 = x1 + w
    y1c = jnp.clip(y1, 0.0, 1.0)
    x1c = jnp.clip(x1, 0.0, 1.0)
    y2c = jnp.clip(y2, 0.0, 1.0)
    x2c = jnp.clip(x2, 0.0, 1.0)
    boxes_ref[...] = jnp.concatenate(
        [y1c[:, None], x1c[:, None], y2c[:, None], x2c[:, None],
         jnp.zeros((y1c.shape[0], 124), jnp.float32)], axis=1)

    dkey_ref[...] = jnp.bitwise_xor(
        lax.bitcast_convert_type(score, jnp.int32), jnp.int32(-1))[:, None]
    cand = (cid > 0) & (score >= MINCONF)
    ccid_ref[...] = jnp.where(cand, cid, 0)[:, None]


def _prep(rois, probs, deltas):
    nb = N // BLK
    return pl.pallas_call(
        _prep_body,
        grid=(nb,),
        in_specs=[
            pl.BlockSpec((BLK, 4), lambda i: (i, 0)),
            pl.BlockSpec((BLK, C), lambda i: (i, 0)),
            pl.BlockSpec((BLK, 4 * C), lambda i: (i, 0)),
        ],
        out_specs=[
            pl.BlockSpec((BLK, 128), lambda i: (i, 0)),
            pl.BlockSpec((BLK, 1), lambda i: (i, 0)),
            pl.BlockSpec((BLK, 1), lambda i: (i, 0)),
        ],
        out_shape=[
            jax.ShapeDtypeStruct((N, 128), jnp.float32),
            jax.ShapeDtypeStruct((N, 1), jnp.int32),
            jax.ShapeDtypeStruct((N, 1), jnp.int32),
        ],
    )(rois, probs, deltas)


def _iota16():
    return lax.broadcasted_iota(jnp.int32, (16,), 0)


def _sc_body(dkey_hbm, ccid_hbm, boxes_hbm, out_hbm,
             keyv, cidv, ak, ai, hist, hist2, cur2, kg, ka, b16, i16, mrowf,
             scnt, sem):
    w = lax.axis_index("s")
    z16i = jnp.zeros((16,), jnp.int32)
    z16f = jnp.zeros((16,), jnp.float32)
    lane0 = _iota16() == 0

    def _sget(ref, off):
        return ref[pl.ds(off, 16)][0]

    def _sput(ref, off, val, dtype=jnp.int32):
        plsc.store_compressed(ref.at[pl.ds(off, 16)],
                              jnp.full((16,), val, dtype), mask=lane0)

    def _sputsel(ref, off, cond, a, b):
        m16 = jnp.full((16,), cond)
        v = jnp.where(m16, jnp.full((16,), a, jnp.int32),
                      jnp.full((16,), b, jnp.int32))
        plsc.store_compressed(ref.at[pl.ds(off, 16)], v, mask=lane0)

    pltpu.sync_copy(dkey_hbm, keyv.at[pl.ds(0, N)])
    pltpu.sync_copy(ccid_hbm, cidv.at[pl.ds(0, N)])

    def _init_scnt(s, _):
        scnt[s] = 0
        return 0
    lax.fori_loop(0, 16, _init_scnt, 0)

    nine = jnp.full((16,), 9.0, jnp.float32)

    def _init_kg(i, _):
        kg[pl.ds(i * 16, 16)] = nine
        return 0
    lax.fori_loop(0, 4 * NCLS * (KCAP // 16), _init_kg, 0)

    def _init_ka(i, _):
        ka[pl.ds(i * 16, 16)] = z16f
        return 0
    lax.fori_loop(0, NCLS * (KCAP // 16), _init_ka, 0)

    def _scan(ch, cur):
        cv = cidv[pl.ds(ch * 16, 16)]
        kv = keyv[pl.ds(ch * 16, 16)]
        mask = (cv != 0) & (((cv - 1) & (NSUB - 1)) == w)
        gidx = ch * 16 + _iota16()
        packed = (gidx << 7) | cv
        plsc.store_compressed(ak.at[pl.ds(cur, 16)], kv, mask=mask)
        plsc.store_compressed(ai.at[pl.ds(cur, 16)], packed, mask=mask)
        return cur + jnp.sum(mask.astype(jnp.int32))

    K = lax.fori_loop(0, N // 16, _scan, jnp.int32(0))

    def _pass(shift, src_k, src_i, dst_k, dst_i):
        ck = (K + 15) >> 4
        base = _iota16() * ck

        def _zh2(i, _):
            l = i >> 4
            b = i & 15
            hist2[l, pl.ds(b * 16, 16)] = z16i
            return 0
        lax.fori_loop(0, 256, _zh2, 0)

        ones16 = jnp.full((16,), 1, jnp.int32)

        def _hv(t, _):
            e = base + t
            valid = e < K
            kv = plsc.load_gather(src_k, [e])
            dv = lax.shift_right_logical(kv, shift) & 255
            plsc.addupdate_scatter(hist2, [_iota16(), dv], ones16, mask=valid)
            return 0
        lax.fori_loop(0, ck, _hv, 0)

        def _tot(c, _):
            def _acc(l, acc):
                return acc + hist2[l, pl.ds(c * 16, 16)]
            hist[pl.ds(c * 16, 16)] = lax.fori_loop(0, 16, _acc, z16i)
            return 0
        lax.fori_loop(0, 16, _tot, 0)

        def _pf(b, run):
            t = _sget(hist, b)
            _sput(hist, b, run)
            return run + t
        lax.fori_loop(0, 256, _pf, jnp.int32(0))

        def _cur(c, _):
            def _row(l, run):
                cur2[l, pl.ds(c * 16, 16)] = run
                return run + hist2[l, pl.ds(c * 16, 16)]
            lax.fori_loop(0, 16, _row, hist[pl.ds(c * 16, 16)])
            return 0
        lax.fori_loop(0, 16, _cur, 0)

        def _sct(t, _):
            e = base + t
            valid = e < K
            kv = plsc.load_gather(src_k, [e])
            iv = plsc.load_gather(src_i, [e])
            dv = lax.shift_right_logical(kv, shift) & 255
            pos = plsc.load_gather(cur2, [_iota16(), dv])
            plsc.store_scatter(dst_k, [pos], kv, mask=valid)
            plsc.store_scatter(dst_i, [pos], iv, mask=valid)
            plsc.addupdate_scatter(cur2, [_iota16(), dv], ones16, mask=valid)
            return 0
        lax.fori_loop(0, ck, _sct, 0)

    _pass(0, ak, ai, keyv, cidv)
    _pass(8, keyv, cidv, ak, ai)
    _pass(16, ak, ai, keyv, cidv)
    sk, si = keyv, cidv

    si[pl.ds(K, 16)] = z16i

    col4 = _iota16() & 3

    @pl.loop(0, (K + 15) >> 4)
    def _chunk(ch):
        i16[...] = lax.shift_right_logical(si[pl.ds(ch * 16, 16)], 7)
        pltpu.async_copy(boxes_hbm.at[i16], b16, sem).wait()
        lim = jnp.minimum(16, K - ch * 16)

        def _elem(l, _):
            t = ch * 16 + l
            p = _sget(si, t)
            c = p & 127
            s = (c - 1) >> 4
            cs = scnt[s]

            @pl.when(cs < MAXI)
            def _():
                g4 = plsc.load_gather(b16, [jnp.full((16,), l, jnp.int32), col4])
                y1 = g4[0]
                x1 = g4[1]
                y2 = g4[2]
                x2 = g4[3]
                ar = (y2 - y1) * (x2 - x1)
                nj = (cs + 15) >> 4

                def _iou(j, acc):
                    o = s * KCAP + j * 16
                    ky1 = kg[pl.ds(o, 16)]
                    kx1 = kg[pl.ds(560 + o, 16)]
                    ky2 = kg[pl.ds(1120 + o, 16)]
                    kx2 = kg[pl.ds(1680 + o, 16)]
                    kar = ka[pl.ds(o, 16)]
                    yy1 = jnp.maximum(y1, ky1)
                    xx1 = jnp.maximum(x1, kx1)
                    yy2 = jnp.minimum(y2, ky2)
                    xx2 = jnp.minimum(x2, kx2)
                    inter = jnp.maximum(0.0, yy2 - yy1) * jnp.maximum(0.0, xx2 - xx1)
                    union = ar + kar - inter
                    iou = inter / jnp.maximum(union, 1e-12)
                    return acc | jnp.any(iou > NMS_T)

                sup = lax.fori_loop(0, nj, _iou, False)

                @pl.when(jnp.logical_not(sup))
                def _():
                    o = s * KCAP + cs
                    _sput(kg, o, y1, jnp.float32)
                    _sput(kg, 560 + o, x1, jnp.float32)
                    _sput(kg, 1120 + o, y2, jnp.float32)
                    _sput(kg, 1680 + o, x2, jnp.float32)
                    _sput(ka, o, ar, jnp.float32)
                    scnt[s] = cs + 1
                    mc = scnt[6]
                    _sput(mrowf, KOFF + mc, _sget(sk, t))
                    _sput(mrowf, GOFF + mc, p)
                    scnt[6] = mc + 1
            return 0

        lax.fori_loop(0, lim, _elem, 0)

    _sput(mrowf, 0, scnt[6])
    pltpu.sync_copy(mrowf.at[pl.ds(0, ROWL)],
                    out_hbm.at[pl.ds(w * ROWL, ROWL)])


def _sc_merge_body(lists_hbm, boxes_hbm, out_hbm,
                   allv, hk, hg, selg, selk, selc, boxr, outv, scnt, ptr, sem):
    ROW = ROWL
    w = lax.axis_index("s")
    z16f = jnp.zeros((16,), jnp.float32)
    lane0 = _iota16() == 0

    def _sput(ref, off, val, dtype=jnp.int32):
        plsc.store_compressed(ref.at[pl.ds(off, 16)],
                              jnp.full((16,), val, dtype), mask=lane0)

    def _ip(s, _):
        ptr[s] = 0
        scnt[s] = 0
        return 0
    lax.fori_loop(0, 16, _ip, 0)

    def _zo(b, _):
        outv[pl.ds(b * 16, 16)] = z16f
        return 0
    lax.fori_loop(0, 39, _zo, 0)

    @pl.when(w == 0)
    def _():
        pltpu.sync_copy(lists_hbm, allv.at[pl.ds(0, 16 * ROW)])

        def _ih(s, _):
            cl = allv[pl.ds(s * ROW, 16)][0]
            m16 = jnp.full((16,), cl > 0)
            kv = jnp.where(m16,
                           jnp.full((16,), allv[pl.ds(s * ROW + KOFF, 16)][0]),
                           jnp.full((16,), SENT))
            gv = jnp.where(m16,
                           jnp.full((16,), allv[pl.ds(s * ROW + GOFF, 16)][0]),
                           jnp.full((16,), BIGP))
            plsc.store_compressed(hk.at[pl.ds(s, 16)], kv, mask=lane0)
            plsc.store_compressed(hg.at[pl.ds(s, 16)], gv, mask=lane0)
            return 0
        lax.fori_loop(0, 16, _ih, 0)

        def _zs(b, _):
            selg[pl.ds(b * 16, 16)] = b * 16 + _iota16()
            return 0
        lax.fori_loop(0, 8, _zs, 0)

        def _pick(t, _):
            hkv = hk[pl.ds(0, 16)]
            m = jnp.min(hkv)

            @pl.when(m != SENT)
            def _():
                hgv = hg[pl.ds(0, 16)]
                cnd = jnp.where(hkv == m, hgv, jnp.full((16,), BIGP))
                gp = jnp.min(cnd)
                l = plsc.all_reduce_ffs(cnd == gp)[0]
                g = lax.shift_right_logical(gp, 7)
                c = gp & 127
                _sput(selg, t, g)
                _sput(selk, t, jnp.bitwise_xor(m, jnp.int32(-1)))
                _sput(selc, t, c)
                scnt[7] = t + 1
                pp = ptr[l] + 1
                ptr[l] = pp
                cl = allv[pl.ds(l * ROW, 16)][0]
                nk = allv[pl.ds(l * ROW + KOFF + pp, 16)][0]
                ng = allv[pl.ds(l * ROW + GOFF + pp, 16)][0]
                m16 = jnp.full((16,), pp < cl)
                kv = jnp.where(m16, jnp.full((16,), nk), jnp.full((16,), SENT))
                gv = jnp.where(m16, jnp.full((16,), ng), jnp.full((16,), BIGP))
                plsc.store_compressed(hk.at[pl.ds(l, 16)], kv, mask=lane0)
                plsc.store_compressed(hg.at[pl.ds(l, 16)], gv, mask=lane0)
            return 0

        lax.fori_loop(0, MAXI, _pick, 0)

        def _conv(t, _):
            sc16 = selc[pl.ds(t, 16)].astype(jnp.float32)
            plsc.store_compressed(outv.at[pl.ds(t * 6 + 4, 16)], sc16,
                                  mask=lane0)
            sk16 = plsc.bitcast(selk[pl.ds(t, 16)], jnp.float32)
            plsc.store_compressed(outv.at[pl.ds(t * 6 + 5, 16)], sk16,
                                  mask=lane0)
            return 0
        lax.fori_loop(0, scnt[7], _conv, 0)

        pltpu.async_copy(boxes_hbm.at[selg], boxr, sem).wait()
        col4 = _iota16() & 3

        def _fill(t, _):
            b4 = plsc.load_gather(boxr, [jnp.full((16,), t, jnp.int32), col4])
            for k in range(4):
                _sput(outv, t * 6 + k, b4[k], jnp.float32)
            return 0
        lax.fori_loop(0, scnt[7], _fill, 0)

        pltpu.sync_copy(outv.at[pl.ds(0, 600)], out_hbm)


KOFF = 8
GOFF = 584
ROWL = 1152


@functools.cache
def _make_sc_nms():
    cp = pltpu.CompilerParams(needs_layout_passes=False,
                              use_tc_tiling_on_sc=False)
    mesh = plsc.VectorSubcoreMesh(core_axis_name="c", subcore_axis_name="s",
                                  num_cores=1)
    nms = pl.kernel(
        _sc_body,
        out_type=jax.ShapeDtypeStruct((16 * ROWL,), jnp.int32),
        mesh=mesh,
        compiler_params=cp,
        scratch_types=[
            pltpu.VMEM((N + 16,), jnp.int32),
            pltpu.VMEM((N + 16,), jnp.int32),
            pltpu.VMEM((N + 16,), jnp.int32),
            pltpu.VMEM((N + 16,), jnp.int32),
            pltpu.VMEM((272,), jnp.int32),
            pltpu.VMEM((16, 272), jnp.int32),
            pltpu.VMEM((16, 272), jnp.int32),
            pltpu.VMEM((2256,), jnp.float32),
            pltpu.VMEM((576,), jnp.float32),
            pltpu.VMEM((16, 128), jnp.float32),
            pltpu.VMEM((16,), jnp.int32),
            pltpu.VMEM((ROWL + 16,), jnp.int32),
            pltpu.SMEM((16,), jnp.int32),
            pltpu.SemaphoreType.DMA,
        ])
    merge = pl.kernel(
        _sc_merge_body,
        out_type=jax.ShapeDtypeStruct((600,), jnp.float32),
        mesh=mesh,
        compiler_params=cp,
        scratch_types=[
            pltpu.VMEM((16 * ROWL + 16,), jnp.int32),
            pltpu.VMEM((32,), jnp.int32),
            pltpu.VMEM((32,), jnp.int32),
            pltpu.VMEM((128,), jnp.int32),
            pltpu.VMEM((128,), jnp.int32),
            pltpu.VMEM((128,), jnp.int32),
            pltpu.VMEM((128, 128), jnp.float32),
            pltpu.VMEM((624,), jnp.float32),
            pltpu.SMEM((16,), jnp.int32),
            pltpu.SMEM((16,), jnp.int32),
            pltpu.SemaphoreType.DMA,
        ])
    return nms, merge


def kernel(ROIs, probs, deltas):
    rois = ROIs.reshape(N, 4)
    p = probs.reshape(N, C)
    d = deltas.reshape(N, 4 * C)
    boxes, dkey, ccid = _prep(rois, p, d)
    nms, merge = _make_sc_nms()
    lists = nms(dkey.reshape(N), ccid.reshape(N), boxes)
    out = merge(lists, boxes)
    return out.reshape(1, MAXI, 6)
